# baseline (speedup 1.0000x reference)
import functools

import jax
import jax.numpy as jnp
from jax import lax
from jax.experimental import pallas as pl
from jax.experimental.pallas import tpu as pltpu
from jax.experimental.pallas import tpu_sc as plsc

NC, NS, LANES = 2, 16, 16
NW = NC * NS

N = 10000
E = 320000
D = 128
G = 64

NPAD = 10240
C = 128
NCH = 160
DH = D // NC
ROWS_PT = NPAD // NS
R = 1000
NBLK = N // R

def _zero_rows(buf, nrows, width):
    zero = jnp.zeros((LANES,), jnp.float32)
    ngrp = width // LANES

    def zr(i, _):
        buf[i // ngrp, pl.ds((i % ngrp) * LANES, LANES)] = zero
        return 0

    lax.fori_loop(0, nrows * ngrp, zr, 0)


def _hist_body(dst_hbm, out_hbm, dst_v, ones_v, bounce_v, accum):
    cid = lax.axis_index("c")
    sid = lax.axis_index("s")
    one = jnp.ones((LANES,), jnp.float32)

    def fill1(i, _):
        ones_v[i, pl.ds(0, LANES)] = one
        return 0

    lax.fori_loop(0, C, fill1, 0)
    _zero_rows(bounce_v, ROWS_PT, LANES)
    pltpu.sync_copy(bounce_v, accum.at[pl.ds(sid * ROWS_PT, ROWS_PT)])
    plsc.subcore_barrier()
    pltpu.sync_copy(dst_hbm.at[sid], dst_v)
    base_ch = cid * (NCH // NC)

    def step(g, _):
        pltpu.sync_copy(ones_v, accum.at[dst_v.at[base_ch + g]], add=True)
        return 0

    lax.fori_loop(0, NCH // NC, step, 0)
    plsc.subcore_barrier()
    pltpu.sync_copy(accum.at[pl.ds(sid * ROWS_PT, ROWS_PT)], bounce_v)
    pltpu.sync_copy(bounce_v, out_hbm.at[cid, pl.ds(sid * ROWS_PT, ROWS_PT)])


@functools.cache
def _hist_kernel():
    mesh = plsc.VectorSubcoreMesh(
        core_axis_name="c", subcore_axis_name="s",
        num_cores=NC, num_subcores=NS)
    return pl.kernel(
        _hist_body,
        out_type=jax.ShapeDtypeStruct((NC, NPAD, LANES), jnp.float32),
        mesh=mesh,
        scratch_types=[
            pltpu.VMEM((NCH + 1, C), jnp.int32),
            pltpu.VMEM((C, LANES), jnp.float32),
            pltpu.VMEM((ROWS_PT, LANES), jnp.float32),
            pltpu.VMEM_SHARED((NPAD, LANES), jnp.float32),
        ],
        compiler_params=pltpu.CompilerParams(use_tc_tiling_on_sc=False),
    )


KBUF = 4


def _agg_body(zs_hbm, src_hbm, dst_hbm, out_hbm,
              src_v, dst_v, bufs, zbuf,
              gs0, gs1, gs2, gs3, ss0, ss1, ss2, ss3, accum):
    gsems = (gs0, gs1, gs2, gs3)
    ssems = (ss0, ss1, ss2, ss3)
    rows_a = bufs.at[0]
    cid = lax.axis_index("c")
    sid = lax.axis_index("s")
    zero = jnp.zeros((LANES,), jnp.float32)
    ngrp = DH // LANES

    def zr(i, _):
        zbuf[i // ngrp, pl.ds((i % ngrp) * LANES, LANES)] = zero
        return 0

    lax.fori_loop(0, C * ngrp, zr, 0)

    def zcp(j, _):
        pltpu.sync_copy(zbuf, accum.at[pl.ds(0, C)])
        return 0

    lax.fori_loop(0, ROWS_PT // C, zcp, 0)
    plsc.subcore_barrier()
    pltpu.sync_copy(src_hbm.at[sid], src_v)
    pltpu.sync_copy(dst_hbm.at[sid], dst_v)


    def step(t, _):
        base = t * KBUF
        gh = [pltpu.async_copy(zs_hbm.at[src_v.at[base + j]], bufs.at[j],
                               gsems[j]) for j in range(KBUF)]
        for j in range(KBUF):
            gh[j].wait()
        return 0

    lax.fori_loop(0, NCH // KBUF, step, 0)
    plsc.subcore_barrier()

    def wcp(j, _):
        off = sid * ROWS_PT + j * C
        pltpu.sync_copy(accum.at[pl.ds(0, C)], zbuf)
        pltpu.sync_copy(zbuf, out_hbm.at[cid, pl.ds(off, C)])
        return 0

    lax.fori_loop(0, ROWS_PT // C, wcp, 0)


@functools.cache
def _agg_kernel():
    mesh = plsc.VectorSubcoreMesh(
        core_axis_name="c", subcore_axis_name="s",
        num_cores=NC, num_subcores=NS)
    return pl.kernel(
        _agg_body,
        out_type=jax.ShapeDtypeStruct((NC, NPAD, DH), jnp.float32),
        mesh=mesh,
        scratch_types=[
            pltpu.VMEM((NCH + 1, C), jnp.int32),
            pltpu.VMEM((NCH + 1, C), jnp.int32),
            pltpu.VMEM((KBUF, C, D), jnp.float32),
            pltpu.VMEM((C, DH), jnp.float32),
            pltpu.SemaphoreType.DMA,
            pltpu.SemaphoreType.DMA,
            pltpu.SemaphoreType.DMA,
            pltpu.SemaphoreType.DMA,
            pltpu.SemaphoreType.DMA,
            pltpu.SemaphoreType.DMA,
            pltpu.SemaphoreType.DMA,
            pltpu.SemaphoreType.DMA,
            pltpu.VMEM_SHARED((C, DH), jnp.float32),
        ],
        compiler_params=pltpu.CompilerParams(use_tc_tiling_on_sc=False),
    )


def _tc1_body(x_ref, w_ref, p0_ref, p1_ref, zs_ref, dn_ref):
    deg = 1.0 + p0_ref[:, 0:1] + p1_ref[:, 0:1]
    dn = lax.rsqrt(deg)
    z = jnp.dot(x_ref[...], w_ref[...], preferred_element_type=jnp.float32)
    zs_ref[...] = z * dn
    dn_ref[...] = jnp.broadcast_to(dn, (R, LANES))


def _tc_layer_body(p0_ref, p1_ref, zsp_ref, dn_ref, b_ref, w_ref, zs_ref):
    dn = dn_ref[:, 0:1]
    agg = jnp.concatenate([p0_ref[...], p1_ref[...]], axis=1)
    h = jnp.maximum(dn * (agg + zsp_ref[...]) + b_ref[...], 0.0)
    zs_ref[...] = dn * jnp.dot(h, w_ref[...],
                               preferred_element_type=jnp.float32)


def _tc_f1_body(p0_ref, p1_ref, zsp_ref, dn_ref, b_ref, gw_ref, batch_ref,
                h3_ref, gate_ref, gmax_ref):
    g = pl.program_id(0)
    dn = dn_ref[:, 0:1]
    agg = jnp.concatenate([p0_ref[...], p1_ref[...]], axis=1)
    h3 = dn * (agg + zsp_ref[...]) + b_ref[...]
    h3_ref[...] = h3
    gate = jnp.dot(h3, gw_ref[...], preferred_element_type=jnp.float32)[:, 0:1]
    gate_ref[...] = jnp.broadcast_to(gate, (R, LANES))

    @pl.when(g == 0)
    def _():
        gmax_ref[...] = jnp.full((8, 128), -jnp.inf, jnp.float32)

    mask = batch_ref[:, 0:1] == lax.broadcasted_iota(jnp.int32, (R, 128), 1)
    gm = jnp.max(jnp.where(mask, gate, -jnp.inf), axis=0, keepdims=True)
    gmax_ref[...] = jnp.maximum(gmax_ref[...], jnp.broadcast_to(gm, (8, 128)))


def _tc_f2_body(h3_ref, gate_ref, batch_ref, gmax_ref, rw_ref, rb_ref,
                out_ref, num_ref, den_ref):
    g = pl.program_id(0)

    @pl.when(g == 0)
    def _():
        num_ref[...] = jnp.zeros((128, 128), jnp.float32)
        den_ref[...] = jnp.zeros((128, 128), jnp.float32)

    mask = batch_ref[:, 0:1] == lax.broadcasted_iota(jnp.int32, (R, 128), 1)
    gmax_row = gmax_ref[0:1, :]
    gmax_node = jnp.max(jnp.where(mask, jnp.broadcast_to(gmax_row, (R, 128)),
                                  -jnp.inf), axis=1, keepdims=True)
    ge = jnp.exp(gate_ref[:, 0:1] - gmax_node)
    wgt = jnp.where(mask, jnp.broadcast_to(ge, (R, 128)), 0.0)
    num_ref[...] += lax.dot_general(
        wgt, h3_ref[...], (((0,), (0,)), ((), ())),
        preferred_element_type=jnp.float32)
    den_ref[...] += lax.dot_general(
        wgt, jnp.ones((R, 128), jnp.float32), (((0,), (0,)), ((), ())),
        preferred_element_type=jnp.float32)

    @pl.when(g == NBLK - 1)
    def _():
        pooled = num_ref[...] / (den_ref[...] + 1e-16)
        r = jnp.dot(pooled, rw_ref[...],
                    preferred_element_type=jnp.float32) + rb_ref[...]
        out_ref[...] = 2.0 * jnp.tanh(r)


def _row_spec(shape):
    return pl.BlockSpec(shape, lambda g: (g, 0))


def _full_spec(shape):
    return pl.BlockSpec(shape, lambda g: (0, 0))


_tc1 = pl.pallas_call(
    _tc1_body,
    grid=(NBLK,),
    in_specs=[_row_spec((R, D)), _full_spec((D, D)),
              _row_spec((R, LANES)), _row_spec((R, LANES))],
    out_specs=[_row_spec((R, D)), _row_spec((R, LANES))],
    out_shape=[jax.ShapeDtypeStruct((N, D), jnp.float32),
               jax.ShapeDtypeStruct((N, LANES), jnp.float32)],
)

_tc_layer = pl.pallas_call(
    _tc_layer_body,
    grid=(NBLK,),
    in_specs=[_row_spec((R, DH)), _row_spec((R, DH)), _row_spec((R, D)),
              _row_spec((R, LANES)), _full_spec((1, D)), _full_spec((D, D))],
    out_specs=_row_spec((R, D)),
    out_shape=jax.ShapeDtypeStruct((N, D), jnp.float32),
)

_tc_f1 = pl.pallas_call(
    _tc_f1_body,
    grid=(NBLK,),
    in_specs=[_row_spec((R, DH)), _row_spec((R, DH)), _row_spec((R, D)),
              _row_spec((R, LANES)), _full_spec((1, D)), _full_spec((D, D)),
              _row_spec((R, LANES))],
    out_specs=[_row_spec((R, D)), _row_spec((R, LANES)),
               _full_spec((8, 128))],
    out_shape=[jax.ShapeDtypeStruct((N, D), jnp.float32),
               jax.ShapeDtypeStruct((N, LANES), jnp.float32),
               jax.ShapeDtypeStruct((8, 128), jnp.float32)],
)

_tc_f2 = pl.pallas_call(
    _tc_f2_body,
    grid=(NBLK,),
    in_specs=[_row_spec((R, D)), _row_spec((R, LANES)),
              _row_spec((R, LANES)), _full_spec((8, 128)),
              _full_spec((D, D)), _full_spec((1, D))],
    out_specs=_full_spec((128, 128)),
    out_shape=jax.ShapeDtypeStruct((128, 128), jnp.float32),
    scratch_shapes=[pltpu.VMEM((128, 128), jnp.float32),
                    pltpu.VMEM((128, 128), jnp.float32)],
)


def kernel(x, edge_index, batch, W1, b1, W2, b2, W3, b3,
           gate_W, gate_b, reg_W, reg_b):
    del gate_b
    src = edge_index[0].astype(jnp.int32)
    dst = edge_index[1].astype(jnp.int32)
    e_pad = NS * (NCH + 1) * C
    npad_e = NS * NCH * C - E
    src_p = jnp.concatenate(
        [src, jnp.zeros((e_pad - E,), jnp.int32)]).reshape(NS, NCH + 1, C)
    dst_p = jnp.concatenate(
        [dst, jnp.full((npad_e,), NPAD - 1, jnp.int32),
         jnp.zeros((e_pad - E - npad_e,), jnp.int32)]).reshape(NS, NCH + 1, C)
    batchb = jnp.broadcast_to(batch.astype(jnp.int32)[:, None], (N, LANES))
    b1r = b1.reshape(1, D)
    b2r = b2.reshape(1, D)
    b3r = b3.reshape(1, D)
    gwp = jnp.pad(gate_W, ((0, 0), (0, D - gate_W.shape[1])))
    rwp = jnp.pad(reg_W, ((0, 0), (0, D - reg_W.shape[1])))
    rbp = jnp.pad(reg_b.reshape(1, -1), ((0, 0), (0, D - reg_b.shape[0])))

    hist = _hist_kernel()(dst_p)
    zs1, dn16 = _tc1(x, W1, hist[0], hist[1])
    agg = _agg_kernel()
    a1 = agg(zs1, src_p, dst_p)
    zs2 = _tc_layer(a1[0, :N], a1[1, :N], zs1, dn16, b1r, W2)
    a2 = agg(zs2, src_p, dst_p)
    zs3 = _tc_layer(a2[0, :N], a2[1, :N], zs2, dn16, b2r, W3)
    a3 = agg(zs3, src_p, dst_p)
    h3, gate16, gmax = _tc_f1(a3[0, :N], a3[1, :N], zs3, dn16, b3r, gwp,
                              batchb)
    out = _tc_f2(h3, gate16, batchb, gmax, rwp, rbp)
    return out[:G, :reg_W.shape[1]]


# speedup vs baseline: 1.9450x; 1.9450x over previous
import functools

import jax
import jax.numpy as jnp
from jax import lax
from jax.experimental import pallas as pl
from jax.experimental.pallas import tpu as pltpu
from jax.experimental.pallas import tpu_sc as plsc

NC, NS, LANES = 2, 16, 16
NW = NC * NS

N = 10000
E = 320000
D = 128
G = 64

NPAD = 10240
C = 128
NCH = 160
DH = D // NC
ROWS_PT = NPAD // NS
R = 1000
NBLK = N // R

def _zero_rows(buf, nrows, width):
    zero = jnp.zeros((LANES,), jnp.float32)
    ngrp = width // LANES

    def zr(i, _):
        buf[i // ngrp, pl.ds((i % ngrp) * LANES, LANES)] = zero
        return 0

    lax.fori_loop(0, nrows * ngrp, zr, 0)


def _hist_body(dst_hbm, out_hbm, dst_v, ones_v, bounce_v, accum):
    cid = lax.axis_index("c")
    sid = lax.axis_index("s")
    one = jnp.ones((LANES,), jnp.float32)

    def fill1(i, _):
        ones_v[i, pl.ds(0, LANES)] = one
        return 0

    lax.fori_loop(0, C, fill1, 0)
    _zero_rows(bounce_v, ROWS_PT, LANES)
    pltpu.sync_copy(bounce_v, accum.at[pl.ds(sid * ROWS_PT, ROWS_PT)])
    plsc.subcore_barrier()
    pltpu.sync_copy(dst_hbm.at[sid], dst_v)
    base_ch = cid * (NCH // NC)

    def step(g, _):
        pltpu.sync_copy(ones_v, accum.at[dst_v.at[base_ch + g]], add=True)
        return 0

    lax.fori_loop(0, NCH // NC, step, 0)
    plsc.subcore_barrier()
    pltpu.sync_copy(accum.at[pl.ds(sid * ROWS_PT, ROWS_PT)], bounce_v)
    pltpu.sync_copy(bounce_v, out_hbm.at[cid, pl.ds(sid * ROWS_PT, ROWS_PT)])


@functools.cache
def _hist_kernel():
    mesh = plsc.VectorSubcoreMesh(
        core_axis_name="c", subcore_axis_name="s",
        num_cores=NC, num_subcores=NS)
    return pl.kernel(
        _hist_body,
        out_type=jax.ShapeDtypeStruct((NC, NPAD, LANES), jnp.float32),
        mesh=mesh,
        scratch_types=[
            pltpu.VMEM((NCH, C), jnp.int32),
            pltpu.VMEM((C, LANES), jnp.float32),
            pltpu.VMEM((ROWS_PT, LANES), jnp.float32),
            pltpu.VMEM_SHARED((NPAD, LANES), jnp.float32),
        ],
        compiler_params=pltpu.CompilerParams(use_tc_tiling_on_sc=False),
    )


KBUF = 4


def _agg_body(zs_hbm, src_hbm, dst_hbm, out_hbm,
              src_v, dst_v, bufs,
              gs0, gs1, gs2, gs3, ss0, ss1, ss2, ss3, accum):
    gsems = (gs0, gs1, gs2, gs3)
    ssems = (ss0, ss1, ss2, ss3)
    rows_a = bufs.at[0]
    cid = lax.axis_index("c")
    sid = lax.axis_index("s")
    zero = jnp.zeros((LANES,), jnp.float32)
    ngrp = DH // LANES

    def zr(i, _):
        bufs[0, i // ngrp, pl.ds((i % ngrp) * LANES, LANES)] = zero
        return 0

    lax.fori_loop(0, C * ngrp, zr, 0)

    def zcp(j, _):
        pltpu.sync_copy(rows_a, accum.at[pl.ds(sid * ROWS_PT + j * C, C)])
        return 0

    lax.fori_loop(0, ROWS_PT // C, zcp, 0)
    plsc.subcore_barrier()
    pltpu.sync_copy(src_hbm.at[sid], src_v)
    pltpu.sync_copy(dst_hbm.at[sid], dst_v)

    def addc(i, _):
        r = i // (C // LANES)
        col = (i % (C // LANES)) * LANES
        src_v[r, pl.ds(col, LANES)] += cid
        return 0

    lax.fori_loop(0, (NCH + 1) * (C // LANES), addc, 0)

    def step(t, _):
        base = t * KBUF
        gh = [pltpu.async_copy(zs_hbm.at[src_v.at[base + j]], bufs.at[j],
                               gsems[j]) for j in range(KBUF)]
        sh = []
        for j in range(KBUF):
            gh[j].wait()
            sh.append(pltpu.async_copy(bufs.at[j],
                                       accum.at[dst_v.at[base + j]],
                                       ssems[j], add=True))
        for j in range(KBUF):
            sh[j].wait()
        return 0

    lax.fori_loop(0, NCH // KBUF, step, 0)
    plsc.subcore_barrier()

    def wcp(j, _):
        off = sid * ROWS_PT + j * C
        pltpu.sync_copy(accum.at[pl.ds(off, C)], rows_a)
        pltpu.sync_copy(rows_a, out_hbm.at[cid, pl.ds(off, C)])
        return 0

    lax.fori_loop(0, ROWS_PT // C, wcp, 0)


@functools.cache
def _agg_kernel():
    mesh = plsc.VectorSubcoreMesh(
        core_axis_name="c", subcore_axis_name="s",
        num_cores=NC, num_subcores=NS)
    return pl.kernel(
        _agg_body,
        out_type=jax.ShapeDtypeStruct((NC, NPAD, DH), jnp.float32),
        mesh=mesh,
        scratch_types=[
            pltpu.VMEM((NCH, C), jnp.int32),
            pltpu.VMEM((NCH, C), jnp.int32),
            pltpu.VMEM((KBUF, C, DH), jnp.float32),
            pltpu.SemaphoreType.DMA,
            pltpu.SemaphoreType.DMA,
            pltpu.SemaphoreType.DMA,
            pltpu.SemaphoreType.DMA,
            pltpu.SemaphoreType.DMA,
            pltpu.SemaphoreType.DMA,
            pltpu.SemaphoreType.DMA,
            pltpu.SemaphoreType.DMA,
            pltpu.VMEM_SHARED((NPAD, DH), jnp.float32),
        ],
        compiler_params=pltpu.CompilerParams(use_tc_tiling_on_sc=False),
    )


def _tc1_body(x_ref, w_ref, p0_ref, p1_ref, zs_ref, dn_ref):
    deg = 1.0 + p0_ref[:, 0:1] + p1_ref[:, 0:1]
    dn = lax.rsqrt(deg)
    z = jnp.dot(x_ref[...], w_ref[...], preferred_element_type=jnp.float32)
    zs_ref[...] = z * dn
    dn_ref[...] = jnp.broadcast_to(dn, (R, LANES))


def _tc_layer_body(p0_ref, p1_ref, zsp_ref, dn_ref, b_ref, w_ref, zs_ref):
    dn = dn_ref[:, 0:1]
    agg = jnp.concatenate([p0_ref[...], p1_ref[...]], axis=1)
    h = jnp.maximum(dn * (agg + zsp_ref[...]) + b_ref[...], 0.0)
    zs_ref[...] = dn * jnp.dot(h, w_ref[...],
                               preferred_element_type=jnp.float32)


def _tc_f1_body(p0_ref, p1_ref, zsp_ref, dn_ref, b_ref, gw_ref, batch_ref,
                h3_ref, gate_ref, gmax_ref):
    g = pl.program_id(0)
    dn = dn_ref[:, 0:1]
    agg = jnp.concatenate([p0_ref[...], p1_ref[...]], axis=1)
    h3 = dn * (agg + zsp_ref[...]) + b_ref[...]
    h3_ref[...] = h3
    gate = jnp.dot(h3, gw_ref[...], preferred_element_type=jnp.float32)[:, 0:1]
    gate_ref[...] = jnp.broadcast_to(gate, (R, LANES))

    @pl.when(g == 0)
    def _():
        gmax_ref[...] = jnp.full((8, 128), -jnp.inf, jnp.float32)

    mask = batch_ref[:, 0:1] == lax.broadcasted_iota(jnp.int32, (R, 128), 1)
    gm = jnp.max(jnp.where(mask, gate, -jnp.inf), axis=0, keepdims=True)
    gmax_ref[...] = jnp.maximum(gmax_ref[...], jnp.broadcast_to(gm, (8, 128)))


def _tc_f2_body(h3_ref, gate_ref, batch_ref, gmax_ref, rw_ref, rb_ref,
                out_ref, num_ref, den_ref):
    g = pl.program_id(0)

    @pl.when(g == 0)
    def _():
        num_ref[...] = jnp.zeros((128, 128), jnp.float32)
        den_ref[...] = jnp.zeros((128, 128), jnp.float32)

    mask = batch_ref[:, 0:1] == lax.broadcasted_iota(jnp.int32, (R, 128), 1)
    gmax_row = gmax_ref[0:1, :]
    gmax_node = jnp.max(jnp.where(mask, jnp.broadcast_to(gmax_row, (R, 128)),
                                  -jnp.inf), axis=1, keepdims=True)
    ge = jnp.exp(gate_ref[:, 0:1] - gmax_node)
    wgt = jnp.where(mask, jnp.broadcast_to(ge, (R, 128)), 0.0)
    num_ref[...] += lax.dot_general(
        wgt, h3_ref[...], (((0,), (0,)), ((), ())),
        preferred_element_type=jnp.float32)
    den_ref[...] += lax.dot_general(
        wgt, jnp.ones((R, 128), jnp.float32), (((0,), (0,)), ((), ())),
        preferred_element_type=jnp.float32)

    @pl.when(g == NBLK - 1)
    def _():
        pooled = num_ref[...] / (den_ref[...] + 1e-16)
        r = jnp.dot(pooled, rw_ref[...],
                    preferred_element_type=jnp.float32) + rb_ref[...]
        out_ref[...] = 2.0 * jnp.tanh(r)


def _row_spec(shape):
    return pl.BlockSpec(shape, lambda g: (g, 0))


def _full_spec(shape):
    return pl.BlockSpec(shape, lambda g: (0, 0))


_tc1 = pl.pallas_call(
    _tc1_body,
    grid=(NBLK,),
    in_specs=[_row_spec((R, D)), _full_spec((D, D)),
              _row_spec((R, LANES)), _row_spec((R, LANES))],
    out_specs=[_row_spec((R, D)), _row_spec((R, LANES))],
    out_shape=[jax.ShapeDtypeStruct((N, D), jnp.float32),
               jax.ShapeDtypeStruct((N, LANES), jnp.float32)],
)

_tc_layer = pl.pallas_call(
    _tc_layer_body,
    grid=(NBLK,),
    in_specs=[_row_spec((R, DH)), _row_spec((R, DH)), _row_spec((R, D)),
              _row_spec((R, LANES)), _full_spec((1, D)), _full_spec((D, D))],
    out_specs=_row_spec((R, D)),
    out_shape=jax.ShapeDtypeStruct((N, D), jnp.float32),
)

_tc_f1 = pl.pallas_call(
    _tc_f1_body,
    grid=(NBLK,),
    in_specs=[_row_spec((R, DH)), _row_spec((R, DH)), _row_spec((R, D)),
              _row_spec((R, LANES)), _full_spec((1, D)), _full_spec((D, D)),
              _row_spec((R, LANES))],
    out_specs=[_row_spec((R, D)), _row_spec((R, LANES)),
               _full_spec((8, 128))],
    out_shape=[jax.ShapeDtypeStruct((N, D), jnp.float32),
               jax.ShapeDtypeStruct((N, LANES), jnp.float32),
               jax.ShapeDtypeStruct((8, 128), jnp.float32)],
)

_tc_f2 = pl.pallas_call(
    _tc_f2_body,
    grid=(NBLK,),
    in_specs=[_row_spec((R, D)), _row_spec((R, LANES)),
              _row_spec((R, LANES)), _full_spec((8, 128)),
              _full_spec((D, D)), _full_spec((1, D))],
    out_specs=_full_spec((128, 128)),
    out_shape=jax.ShapeDtypeStruct((128, 128), jnp.float32),
    scratch_shapes=[pltpu.VMEM((128, 128), jnp.float32),
                    pltpu.VMEM((128, 128), jnp.float32)],
)


def kernel(x, edge_index, batch, W1, b1, W2, b2, W3, b3,
           gate_W, gate_b, reg_W, reg_b):
    del gate_b
    src = edge_index[0].astype(jnp.int32)
    dst = edge_index[1].astype(jnp.int32)
    e_pad = NS * NCH * C
    src_p = jnp.concatenate(
        [src * 2, jnp.zeros((e_pad - E,), jnp.int32)]).reshape(NS, NCH, C)
    dst_p = jnp.concatenate(
        [dst, jnp.full((e_pad - E,), NPAD - 1, jnp.int32)]).reshape(NS, NCH, C)
    batchb = jnp.broadcast_to(batch.astype(jnp.int32)[:, None], (N, LANES))
    b1r = b1.reshape(1, D)
    b2r = b2.reshape(1, D)
    b3r = b3.reshape(1, D)
    gwp = jnp.pad(gate_W, ((0, 0), (0, D - gate_W.shape[1])))
    rwp = jnp.pad(reg_W, ((0, 0), (0, D - reg_W.shape[1])))
    rbp = jnp.pad(reg_b.reshape(1, -1), ((0, 0), (0, D - reg_b.shape[0])))

    hist = _hist_kernel()(dst_p)
    zs1, dn16 = _tc1(x, W1, hist[0], hist[1])
    agg = _agg_kernel()
    a1 = agg(zs1.reshape(N * NC, DH), src_p, dst_p)
    zs2 = _tc_layer(a1[0, :N], a1[1, :N], zs1, dn16, b1r, W2)
    a2 = agg(zs2.reshape(N * NC, DH), src_p, dst_p)
    zs3 = _tc_layer(a2[0, :N], a2[1, :N], zs2, dn16, b2r, W3)
    a3 = agg(zs3.reshape(N * NC, DH), src_p, dst_p)
    h3, gate16, gmax = _tc_f1(a3[0, :N], a3[1, :N], zs3, dn16, b3r, gwp,
                              batchb)
    out = _tc_f2(h3, gate16, batchb, gmax, rwp, rbp)
    return out[:G, :reg_W.shape[1]]
